# baseline (speedup 1.0000x reference)
import functools

import jax
import jax.numpy as jnp
from jax import lax
from jax.experimental import pallas as pl
from jax.experimental.pallas import tpu as pltpu
from jax.experimental.pallas import tpu_sc as plsc

_R = 2.0
_T = 1.0
_NC = 2
_NS = 16
_NW = _NC * _NS
_C = 80
_L = 16
_DIM = 128


def _normalize_body(h_ref, o_ref):
    h = h_ref[...]
    ss = jnp.sum(h * h, axis=1, keepdims=True)
    norm = jnp.sqrt(ss)
    scale = jnp.minimum(1.0, 1.0 / jnp.maximum(norm, 1e-12))
    o_ref[...] = h * scale


def _normalize(h):
    return pl.pallas_call(
        _normalize_body,
        out_shape=jax.ShapeDtypeStruct(h.shape, h.dtype),
    )(h)


@functools.lru_cache(maxsize=None)
def _sc_decode(n_edges, n_nodes):
    ew = n_edges // _NW
    nchunks = ew // _C
    nquads = nchunks // 4
    mesh = plsc.VectorSubcoreMesh(core_axis_name="c", subcore_axis_name="s")

    @functools.partial(
        pl.kernel,
        out_type=jax.ShapeDtypeStruct((n_edges,), jnp.float32),
        mesh=mesh,
        compiler_params=pltpu.CompilerParams(needs_layout_passes=False),
        scratch_types=[
            pltpu.VMEM((4, _C), jnp.int32),
            pltpu.VMEM((4, _C), jnp.int32),
            pltpu.VMEM((4, _C, _DIM), jnp.float32),
            pltpu.VMEM((4, _C, _DIM), jnp.float32),
            pltpu.VMEM((_C, _L + 1), jnp.float32),
            pltpu.VMEM((2, _C), jnp.float32),
        ] + [pltpu.SemaphoreType.DMA] * 14,
    )
    def k(tab_hbm, idx0_hbm, idx1_hbm, out_hbm,
          idx0_r, idx1_r, rows_a, rows_b, sq, outv,
          sem_i0, sem_i1, sem_i2, sem_i3,
          sem_a0, sem_a1, sem_a2, sem_a3,
          sem_b0, sem_b1, sem_b2, sem_b3, sem_o0, sem_o1):
        cid = lax.axis_index("c")
        sid = lax.axis_index("s")
        wid = sid * _NC + cid
        base0 = wid * ew
        sems_i = (sem_i0, sem_i1, sem_i2, sem_i3)
        sems_a = (sem_a0, sem_a1, sem_a2, sem_a3)
        sems_b = (sem_b0, sem_b1, sem_b2, sem_b3)
        sems_o = (sem_o0, sem_o1)

        def idx_fetch(g, slot):
            src0 = idx0_hbm.at[pl.ds(base0 + g * _C, _C)]
            src1 = idx1_hbm.at[pl.ds(base0 + g * _C, _C)]
            pltpu.async_copy(src0, idx0_r.at[slot], sems_i[slot])
            pltpu.async_copy(src1, idx1_r.at[slot], sems_i[slot])

        def idx_wait(slot):
            cp = pltpu.make_async_copy(
                idx0_hbm.at[pl.ds(base0, _C)], idx0_r.at[slot], sems_i[slot])
            cp.wait()
            cp.wait()

        def start_gather(slot, b):
            pltpu.async_copy(tab_hbm.at[idx0_r.at[slot]], rows_a.at[b], sems_a[b])
            pltpu.async_copy(tab_hbm.at[idx1_r.at[slot]], rows_b.at[b], sems_b[b])

        def wait_gather(b):
            pltpu.make_async_copy(
                tab_hbm.at[idx0_r.at[0]], rows_a.at[b], sems_a[b]).wait()
            pltpu.make_async_copy(
                tab_hbm.at[idx1_r.at[0]], rows_b.at[b], sems_b[b]).wait()

        def out_write(g, b):
            pltpu.async_copy(
                outv.at[b], out_hbm.at[pl.ds(base0 + g * _C, _C)], sems_o[b])

        def out_wait(b):
            pltpu.make_async_copy(
                outv.at[b], out_hbm.at[pl.ds(base0, _C)], sems_o[b]).wait()

        def phase_a(b):
            @plsc.parallel_loop(0, _C, 1, unroll=8)
            def _(e):
                accs = [jnp.zeros((_L,), jnp.float32) for _ in range(2)]
                for kk in range(_DIM // _L):
                    va = rows_a[b, e, pl.ds(kk * _L, _L)]
                    vb = rows_b[b, e, pl.ds(kk * _L, _L)]
                    d = va - vb
                    accs[kk % 2] = accs[kk % 2] + d * d
                sq[e, pl.ds(0, _L)] = accs[0] + accs[1]

        def phase_b(b2):
            for gg in range(_C // _L):
                e0 = gg * _L
                rows = e0 + lax.iota(jnp.int32, _L)
                tots = [jnp.zeros((_L,), jnp.float32) for _ in range(4)]
                for col in range(_L):
                    cols = jnp.full((_L,), col, jnp.int32)
                    tots[col % 4] = tots[col % 4] + plsc.load_gather(
                        sq, [rows, cols])
                tot = (tots[0] + tots[1]) + (tots[2] + tots[3])
                p = 1.0 / (1.0 + jnp.exp((tot - _R) / _T))
                outv[b2, pl.ds(e0, _L)] = p

        for j in range(4):
            idx_fetch(j, j)
        for j in range(3):
            idx_wait(j)
            start_gather(j, j)

        def quad_body(q, carry):
            g0 = 4 * q
            for j in range(4):
                b = j
                b2 = j % 2
                g = g0 + j
                wait_gather(b)

                @pl.when(g >= 2)
                def _():
                    out_wait(b2)

                phase_a(b)

                @pl.when(g + 3 < nchunks)
                def _():
                    idx_wait((j + 3) % 4)
                    start_gather((j + 3) % 4, (b + 3) % 4)

                @pl.when(g + 4 < nchunks)
                def _():
                    idx_fetch(g + 4, j)

                phase_b(b2)
                out_write(g, b2)

            return carry

        lax.fori_loop(0, nquads, quad_body, 0)

        for g in range(nquads * 4, nchunks):
            b = g % 4
            b2 = g % 2
            wait_gather(b)
            out_wait(b2)
            phase_a(b)
            phase_b(b2)
            out_write(g, b2)

        out_wait(0)
        out_wait(1)

    return k


def kernel(h, idx):
    hn = _normalize(h)
    idx0 = idx[:, 0]
    idx1 = idx[:, 1]
    return _sc_decode(idx.shape[0], h.shape[0])(hn, idx0, idx1)


# speedup vs baseline: 1.0618x; 1.0618x over previous
import functools

import jax
import jax.numpy as jnp
from jax import lax
from jax.experimental import pallas as pl
from jax.experimental.pallas import tpu as pltpu
from jax.experimental.pallas import tpu_sc as plsc

_R = 2.0
_T = 1.0
_NC = 2
_NS = 16
_NW = _NC * _NS
_C = 80
_L = 16
_DIM = 128


def _normalize_body(h_ref, o_ref):
    h = h_ref[...]
    ss = jnp.sum(h * h, axis=1, keepdims=True)
    norm = jnp.sqrt(ss)
    scale = jnp.minimum(1.0, 1.0 / jnp.maximum(norm, 1e-12))
    o_ref[...] = h * scale


def _normalize(h):
    return pl.pallas_call(
        _normalize_body,
        out_shape=jax.ShapeDtypeStruct(h.shape, h.dtype),
    )(h)


@functools.lru_cache(maxsize=None)
def _sc_decode(n_edges, n_nodes):
    ew = n_edges // _NW
    nchunks = ew // _C
    nquads = nchunks // 4
    mesh = plsc.VectorSubcoreMesh(core_axis_name="c", subcore_axis_name="s")

    @functools.partial(
        pl.kernel,
        out_type=jax.ShapeDtypeStruct((n_edges,), jnp.float32),
        mesh=mesh,
        compiler_params=pltpu.CompilerParams(needs_layout_passes=False),
        scratch_types=[
            pltpu.VMEM((4, _C), jnp.int32),
            pltpu.VMEM((4, _C), jnp.int32),
            pltpu.VMEM((4, _C, _DIM), jnp.float32),
            pltpu.VMEM((4, _C, _DIM), jnp.float32),
            pltpu.VMEM((_C, _L + 1), jnp.float32),
            pltpu.VMEM((2, _C), jnp.float32),
        ] + [pltpu.SemaphoreType.DMA] * 14,
    )
    def k(tab_hbm, idx0_hbm, idx1_hbm, out_hbm,
          idx0_r, idx1_r, rows_a, rows_b, sq, outv,
          sem_i0, sem_i1, sem_i2, sem_i3,
          sem_a0, sem_a1, sem_a2, sem_a3,
          sem_b0, sem_b1, sem_b2, sem_b3, sem_o0, sem_o1):
        cid = lax.axis_index("c")
        sid = lax.axis_index("s")
        wid = sid * _NC + cid
        base0 = wid * ew
        sems_i = (sem_i0, sem_i1, sem_i2, sem_i3)
        sems_a = (sem_a0, sem_a1, sem_a2, sem_a3)
        sems_b = (sem_b0, sem_b1, sem_b2, sem_b3)
        sems_o = (sem_o0, sem_o1)

        def idx_fetch(g, slot):
            src0 = idx0_hbm.at[pl.ds(base0 + g * _C, _C)]
            src1 = idx1_hbm.at[pl.ds(base0 + g * _C, _C)]
            pltpu.async_copy(src0, idx0_r.at[slot], sems_i[slot])
            pltpu.async_copy(src1, idx1_r.at[slot], sems_i[slot])

        def idx_wait(slot):
            cp = pltpu.make_async_copy(
                idx0_hbm.at[pl.ds(base0, _C)], idx0_r.at[slot], sems_i[slot])
            cp.wait()
            cp.wait()

        def start_gather(slot, b):
            pltpu.async_copy(tab_hbm.at[idx0_r.at[slot]], rows_a.at[b], sems_a[b])
            pltpu.async_copy(tab_hbm.at[idx1_r.at[slot]], rows_b.at[b], sems_b[b])

        def wait_gather(b):
            pltpu.make_async_copy(
                tab_hbm.at[idx0_r.at[0]], rows_a.at[b], sems_a[b]).wait()
            pltpu.make_async_copy(
                tab_hbm.at[idx1_r.at[0]], rows_b.at[b], sems_b[b]).wait()

        def out_write(g, b):
            pltpu.async_copy(
                outv.at[b], out_hbm.at[pl.ds(base0 + g * _C, _C)], sems_o[b])

        def out_wait(b):
            pltpu.make_async_copy(
                outv.at[b], out_hbm.at[pl.ds(base0, _C)], sems_o[b]).wait()

        def phase_a(b):
            @plsc.parallel_loop(0, _C, 1, unroll=4)
            def _(e):
                accs = [jnp.zeros((_L,), jnp.float32) for _ in range(4)]
                for kk in range(_DIM // _L):
                    va = rows_a[b, e, pl.ds(kk * _L, _L)]
                    vb = rows_b[b, e, pl.ds(kk * _L, _L)]
                    d = va - vb
                    accs[kk % 4] = accs[kk % 4] + d * d
                sq[e, pl.ds(0, _L)] = (accs[0] + accs[1]) + (accs[2] + accs[3])

        def phase_b(b2):
            for gg in range(_C // _L):
                e0 = gg * _L
                rows = e0 + lax.iota(jnp.int32, _L)
                tots = [jnp.zeros((_L,), jnp.float32) for _ in range(4)]
                for col in range(_L):
                    cols = jnp.full((_L,), col, jnp.int32)
                    tots[col % 4] = tots[col % 4] + plsc.load_gather(
                        sq, [rows, cols])
                tot = (tots[0] + tots[1]) + (tots[2] + tots[3])
                p = 1.0 / (1.0 + jnp.exp((tot - _R) / _T))
                outv[b2, pl.ds(e0, _L)] = p

        for j in range(4):
            idx_fetch(j, j)
        for j in range(3):
            idx_wait(j)
            start_gather(j, j)

        def quad_body(q, carry):
            g0 = 4 * q
            for j in range(4):
                b = j
                b2 = j % 2
                g = g0 + j
                wait_gather(b)

                @pl.when(g >= 2)
                def _():
                    out_wait(b2)

                phase_a(b)

                @pl.when(g + 3 < nchunks)
                def _():
                    idx_wait((j + 3) % 4)
                    start_gather((j + 3) % 4, (b + 3) % 4)

                @pl.when(g + 4 < nchunks)
                def _():
                    idx_fetch(g + 4, j)

                phase_b(b2)
                out_write(g, b2)

            return carry

        lax.fori_loop(0, nquads, quad_body, 0)

        for g in range(nquads * 4, nchunks):
            b = g % 4
            b2 = g % 2
            wait_gather(b)
            out_wait(b2)
            phase_a(b)
            phase_b(b2)
            out_write(g, b2)

        out_wait(0)
        out_wait(1)

    return k


def kernel(h, idx):
    hn = _normalize(h)
    idx0 = idx[:, 0]
    idx1 = idx[:, 1]
    return _sc_decode(idx.shape[0], h.shape[0])(hn, idx0, idx1)


# speedup vs baseline: 1.0634x; 1.0015x over previous
import functools

import jax
import jax.numpy as jnp
from jax import lax
from jax.experimental import pallas as pl
from jax.experimental.pallas import tpu as pltpu
from jax.experimental.pallas import tpu_sc as plsc

_R = 2.0
_T = 1.0
_NC = 2
_NS = 16
_NW = _NC * _NS
_C = 80
_L = 16
_DIM = 128


def _normalize_body(h_ref, o_ref):
    h = h_ref[...]
    ss = jnp.sum(h * h, axis=1, keepdims=True)
    norm = jnp.sqrt(ss)
    scale = jnp.minimum(1.0, 1.0 / jnp.maximum(norm, 1e-12))
    o_ref[...] = h * scale


def _normalize(h):
    return pl.pallas_call(
        _normalize_body,
        out_shape=jax.ShapeDtypeStruct(h.shape, h.dtype),
    )(h)


@functools.lru_cache(maxsize=None)
def _sc_decode(n_edges, n_nodes):
    ew = n_edges // _NW
    nchunks = ew // _C
    nquads = nchunks // 4
    mesh = plsc.VectorSubcoreMesh(core_axis_name="c", subcore_axis_name="s")

    @functools.partial(
        pl.kernel,
        out_type=jax.ShapeDtypeStruct((n_edges,), jnp.float32),
        mesh=mesh,
        compiler_params=pltpu.CompilerParams(needs_layout_passes=False),
        scratch_types=[
            pltpu.VMEM((4, _C), jnp.int32),
            pltpu.VMEM((4, _C), jnp.int32),
            pltpu.VMEM((4, _C, _DIM), jnp.float32),
            pltpu.VMEM((4, _C, _DIM), jnp.float32),
            pltpu.VMEM((_C, _L + 1), jnp.float32),
            pltpu.VMEM((2, _C), jnp.float32),
        ] + [pltpu.SemaphoreType.DMA] * 14,
    )
    def k(tab_hbm, idx0_hbm, idx1_hbm, out_hbm,
          idx0_r, idx1_r, rows_a, rows_b, sq, outv,
          sem_i0, sem_i1, sem_i2, sem_i3,
          sem_a0, sem_a1, sem_a2, sem_a3,
          sem_b0, sem_b1, sem_b2, sem_b3, sem_o0, sem_o1):
        cid = lax.axis_index("c")
        sid = lax.axis_index("s")
        wid = sid * _NC + cid
        base0 = wid * ew
        sems_i = (sem_i0, sem_i1, sem_i2, sem_i3)
        sems_a = (sem_a0, sem_a1, sem_a2, sem_a3)
        sems_b = (sem_b0, sem_b1, sem_b2, sem_b3)
        sems_o = (sem_o0, sem_o1)

        def idx_fetch(g, slot):
            src0 = idx0_hbm.at[pl.ds(base0 + g * _C, _C)]
            src1 = idx1_hbm.at[pl.ds(base0 + g * _C, _C)]
            pltpu.async_copy(src0, idx0_r.at[slot], sems_i[slot])
            pltpu.async_copy(src1, idx1_r.at[slot], sems_i[slot])

        def idx_wait(slot):
            cp = pltpu.make_async_copy(
                idx0_hbm.at[pl.ds(base0, _C)], idx0_r.at[slot], sems_i[slot])
            cp.wait()
            cp.wait()

        def start_gather(slot, b):
            pltpu.async_copy(tab_hbm.at[idx0_r.at[slot]], rows_a.at[b], sems_a[b])
            pltpu.async_copy(tab_hbm.at[idx1_r.at[slot]], rows_b.at[b], sems_b[b])

        def wait_gather(b):
            pltpu.make_async_copy(
                tab_hbm.at[idx0_r.at[0]], rows_a.at[b], sems_a[b]).wait()
            pltpu.make_async_copy(
                tab_hbm.at[idx1_r.at[0]], rows_b.at[b], sems_b[b]).wait()

        def out_write(g, b):
            pltpu.async_copy(
                outv.at[b], out_hbm.at[pl.ds(base0 + g * _C, _C)], sems_o[b])

        def out_wait(b):
            pltpu.make_async_copy(
                outv.at[b], out_hbm.at[pl.ds(base0, _C)], sems_o[b]).wait()

        def phase_a(b):
            @plsc.parallel_loop(0, _C, 1, unroll=4)
            def _(e):
                sqs = []
                for kk in range(_DIM // _L):
                    va = rows_a[b, e, pl.ds(kk * _L, _L)]
                    vb = rows_b[b, e, pl.ds(kk * _L, _L)]
                    d = va - vb
                    sqs.append(d * d)
                s01 = sqs[0] + sqs[1]
                s23 = sqs[2] + sqs[3]
                s45 = sqs[4] + sqs[5]
                s67 = sqs[6] + sqs[7]
                sq[e, pl.ds(0, _L)] = (s01 + s23) + (s45 + s67)

        def phase_b(b2):
            for gg in range(_C // _L):
                e0 = gg * _L
                rows = e0 + lax.iota(jnp.int32, _L)
                tots = [jnp.zeros((_L,), jnp.float32) for _ in range(4)]
                for col in range(_L):
                    cols = jnp.full((_L,), col, jnp.int32)
                    tots[col % 4] = tots[col % 4] + plsc.load_gather(
                        sq, [rows, cols])
                tot = (tots[0] + tots[1]) + (tots[2] + tots[3])
                p = 1.0 / (1.0 + jnp.exp((tot - _R) / _T))
                outv[b2, pl.ds(e0, _L)] = p

        for j in range(4):
            idx_fetch(j, j)
        for j in range(3):
            idx_wait(j)
            start_gather(j, j)

        def quad_body(q, carry):
            g0 = 4 * q
            for j in range(4):
                b = j
                b2 = j % 2
                g = g0 + j
                wait_gather(b)

                @pl.when(g >= 2)
                def _():
                    out_wait(b2)

                phase_a(b)

                @pl.when(g + 3 < nchunks)
                def _():
                    idx_wait((j + 3) % 4)
                    start_gather((j + 3) % 4, (b + 3) % 4)

                @pl.when(g + 4 < nchunks)
                def _():
                    idx_fetch(g + 4, j)

                phase_b(b2)
                out_write(g, b2)

            return carry

        lax.fori_loop(0, nquads, quad_body, 0)

        for g in range(nquads * 4, nchunks):
            b = g % 4
            b2 = g % 2
            wait_gather(b)
            out_wait(b2)
            phase_a(b)
            phase_b(b2)
            out_write(g, b2)

        out_wait(0)
        out_wait(1)

    return k


def kernel(h, idx):
    hn = _normalize(h)
    idx0 = idx[:, 0]
    idx1 = idx[:, 1]
    return _sc_decode(idx.shape[0], h.shape[0])(hn, idx0, idx1)
